# baseline (speedup 1.0000x reference)
import jax
import jax.numpy as jnp
from jax import lax
from jax.experimental import pallas as pl

_K = 32
_BITS_TWO = 0x40000000
_BITS_INF = 0x7F800000
_N = 32768
_ROWS, _COLS = 256, 128


def _body(x_ref, y_ref):
    x = x_ref[...]
    ones_row = jnp.ones((1, _ROWS), jnp.float32)

    def count_ge(mask_f):
        return jnp.sum(mask_f)

    ones128 = jnp.ones((_COLS, _COLS), jnp.float32)

    def count_ge_vec(mask_f):
        part = jnp.sum(mask_f, axis=0, keepdims=True)
        return lax.dot_general(part, ones128, (((1,), (0,)), ((), ())),
                               preferred_element_type=jnp.float32)

    kf = jnp.full((1, _COLS), float(_K), jnp.float32)

    def val_step(_, lohi):
        lo, hi = lohi
        span = hi - lo
        t1 = lo + span // 4
        t2 = lo + span // 2
        t3 = lo + span // 2 + span // 4
        c1 = count_ge_vec(jnp.where(
            x >= lax.bitcast_convert_type(t1, jnp.float32), 1.0, 0.0))
        c2 = count_ge_vec(jnp.where(
            x >= lax.bitcast_convert_type(t2, jnp.float32), 1.0, 0.0))
        c3 = count_ge_vec(jnp.where(
            x >= lax.bitcast_convert_type(t3, jnp.float32), 1.0, 0.0))
        b3 = c3 >= kf
        b2 = c2 >= kf
        b1 = c1 >= kf
        nlo = jnp.where(b3, t3, jnp.where(b2, t2, jnp.where(b1, t1, lo)))
        nhi = jnp.where(b3, hi, jnp.where(b2, t3, jnp.where(b1, t2, t1)))
        return nlo, nhi

    lo0 = jnp.full((1, _COLS), _BITS_TWO, jnp.int32)
    hi0 = jnp.full((1, _COLS), _BITS_INF, jnp.int32)
    lo_v, _ = lax.fori_loop(0, 17, val_step, (lo0, hi0))
    t = jnp.max(lo_v)
    t_f = lax.bitcast_convert_type(t, jnp.float32)

    gt = x > t_f
    c_gt = count_ge(jnp.where(gt, 1.0, 0.0)).astype(jnp.int32)
    m = jnp.where(t == _BITS_TWO, 0, _K - c_gt)

    idx = lax.broadcasted_iota(jnp.int32, (_ROWS, _COLS), 0) * _COLS + \
        lax.broadcasted_iota(jnp.int32, (_ROWS, _COLS), 1)
    tie = x == t_f
    tie_f = jnp.where(tie, 1.0, 0.0)

    def one_tie():
        return jnp.min(jnp.where(tie, idx, jnp.int32(_N))) + 1

    def multi_tie():
        def idx_step(_, lohi):
            lo2, hi2 = lohi
            mid = lo2 + (hi2 - lo2) // 2
            c = count_ge(jnp.where(idx < mid, tie_f, 0.0)).astype(jnp.int32)
            small = c < m
            return jnp.where(small, mid, lo2), jnp.where(small, hi2, mid)

        _, cut = lax.fori_loop(0, 15, idx_step, (jnp.int32(0), jnp.int32(_N)))
        return cut

    cut = lax.cond(m <= 1, one_tie, multi_tie)

    win = gt | (tie & (idx < cut) & (m > 0))
    y_ref[...] = jnp.where(win, 1.0, 0.0).astype(jnp.float32)


def kernel(x):
    x2 = x.reshape(_ROWS, _COLS)
    y = pl.pallas_call(
        _body,
        out_shape=jax.ShapeDtypeStruct((_ROWS, _COLS), jnp.float32),
    )(x2)
    return y.reshape(_N)
